# baseline (speedup 1.0000x reference)
import functools

import jax
import jax.numpy as jnp
from jax import lax
from jax.experimental import pallas as pl
from jax.experimental.pallas import tpu as pltpu
from jax.experimental.pallas import tpu_sc as plsc

N = 10000
L = 16
NW = 32
W = 320
NV = W // L
HALF = NV // 2
RB = 125
NBLK = N // RB


def _make_sc_kernel():
    mesh = plsc.VectorSubcoreMesh(core_axis_name="c", subcore_axis_name="s")

    @functools.partial(
        pl.kernel,
        out_type=(
            jax.ShapeDtypeStruct((1, N), jnp.float32),
            jax.ShapeDtypeStruct((N,), jnp.int32),
        ),
        mesh=mesh,
        compiler_params=pltpu.CompilerParams(use_tc_tiling_on_sc=False,
                                             needs_layout_passes=False),
        scratch_types=[
            pltpu.VMEM((N,), jnp.float32),
            pltpu.VMEM((RB, W), jnp.float32),
            pltpu.VMEM((RB, W), jnp.float32),
            pltpu.VMEM((W,), jnp.float32),
            pltpu.VMEM((W,), jnp.int32),
            pltpu.SemaphoreType.DMA,
            pltpu.SemaphoreType.DMA,
        ],
    )
    def sc_kernel(e_hbm, r_hbm, out_emb, out_idx,
                  e_v, buf0, buf1, val_v, idx_v, sem0, sem1):
        cid = lax.axis_index("c")
        sid = lax.axis_index("s")
        w = sid * 2 + cid
        c0 = pl.multiple_of((w * (N - W) // (NW - 1) // 8) * 8, 8)

        pltpu.sync_copy(e_hbm.at[0], e_v)

        for j in range(NV):
            val_v[pl.ds(j * L, L)] = jnp.zeros((L,), jnp.float32)
            idx_v[pl.ds(j * L, L)] = jnp.zeros((L,), jnp.int32)

        bufs = (buf0, buf1)
        sems = (sem0, sem1)

        def start(b, k):
            pltpu.async_copy(
                r_hbm.at[pl.ds(b * RB, RB), pl.ds(c0, W)], bufs[k], sems[k])

        def wait(b, k):
            pltpu.make_async_copy(
                r_hbm.at[pl.ds(b * RB, RB), pl.ds(c0, W)], bufs[k],
                sems[k]).wait()

        start(0, 0)
        start(1, 1)

        def process(b, buf):
            base = b * RB
            for h in range(2):
                off = h * HALF * L
                carry = tuple(
                    val_v[pl.ds(off + j * L, L)] for j in range(HALF)
                ) + tuple(
                    idx_v[pl.ds(off + j * L, L)] for j in range(HALF)
                )

                def row_body(r, cr, off=off, base=base, buf=buf):
                    vals = list(cr[:HALF])
                    idxs = list(cr[HALF:])
                    ivec = jnp.full((L,), base + r, jnp.int32)
                    ev = plsc.load_gather(e_v, [ivec])
                    for j in range(HALF):
                        prod = buf[r, pl.ds(off + j * L, L)] * ev
                        m = prod > vals[j]
                        vals[j] = jnp.where(m, prod, vals[j])
                        idxs[j] = jnp.where(m, ivec, idxs[j])
                    return tuple(vals) + tuple(idxs)

                carry = lax.fori_loop(0, RB, row_body, carry)
                for j in range(HALF):
                    val_v[pl.ds(off + j * L, L)] = carry[j]
                    idx_v[pl.ds(off + j * L, L)] = carry[HALF + j]

        def outer(g, acc):
            for k in range(2):
                b = 2 * g + k
                wait(b, k)
                process(b, bufs[k])

                @pl.when(b + 2 < NBLK)
                def _(b=b, k=k):
                    start(b + 2, k)
            return acc

        lax.fori_loop(0, NBLK // 2, outer, 0)

        pltpu.sync_copy(val_v, out_emb.at[0, pl.ds(c0, W)])
        pltpu.sync_copy(idx_v, out_idx.at[pl.ds(c0, W)])

    return sc_kernel


_sc_kernel = _make_sc_kernel()


@jax.jit
def kernel(embedding, r_embedding):
    new_embedding, r_argmax = _sc_kernel(embedding, r_embedding)
    return new_embedding, r_argmax
